# baseline (speedup 1.0000x reference)
import functools

import jax
import jax.numpy as jnp
from jax import lax
from jax.experimental import pallas as pl
from jax.experimental.pallas import tpu as pltpu
from jax.experimental.pallas import tpu_sc as plsc

NUM_EMB = 1000000
DIM = 32
BATCH = 16384
HIST = 50
NC = 2
NS = 16
NW = NC * NS
TBW = BATCH // 128 // NW
NBLK = HIST * TBW
L = 16

BLK = 128
NBL = NUM_EMB // BLK
TAIL0 = NBL * BLK
TAILN = NUM_EMB - TAIL0
ABPW = -(-NBL // NW)


def _detile_block(tin, r2, r3, n):
    for d in range(DIM):
        ds_ = jnp.full((L,), d, jnp.int32)
        for k in range(n // L):
            ids = lax.iota(jnp.int32, L) + L * k
            v = tin[d, pl.ds(L * k, L)]
            plsc.store_scatter(r2, [ids, ds_], v)
    for i in range(n):
        q, r, c = i >> 5, (i >> 2) & 7, 32 * (i & 3)
        r3[q, r, pl.ds(c, L)] = r2[i, pl.ds(0, L)]
        r3[q, r, pl.ds(c + L, L)] = r2[i, pl.ds(L, L)]


def _body_a(tbl_hbm, out_hbm, t0, t1, r20, r21, r30, r31, ttin, tr2, tr3,
            is0, is1, os0, os1):
    wid = lax.axis_index("s") * NC + lax.axis_index("c")
    tins, r2s, r3s = (t0, t1), (r20, r21), (r30, r31)
    isems, osems = (is0, is1), (os0, os1)

    def blk(j):
        return wid * ABPW + j

    def fire_in(j, s):
        pltpu.async_copy(
            tbl_hbm.at[:, pl.ds(BLK * blk(j), BLK)], tins[s], isems[s]
        )

    def wait_in(s):
        pltpu.make_async_copy(
            tbl_hbm.at[:, pl.ds(0, BLK)], tins[s], isems[s]
        ).wait()

    def fire_out(j, s):
        pltpu.async_copy(r3s[s], out_hbm.at[pl.ds(4 * blk(j), 4)], osems[s])

    def wait_out(s):
        pltpu.make_async_copy(
            r3s[s], out_hbm.at[pl.ds(0, 4)], osems[s]
        ).wait()

    @pl.when(blk(0) < NBL)
    def _p0():
        fire_in(0, 0)

    @pl.when(blk(1) < NBL)
    def _p1():
        fire_in(1, 1)

    def step(j, _):
        for s in range(2):
            @pl.when(blk(j + s) < NBL)
            def _do():
                @pl.when(j >= 2)
                def _free():
                    wait_out(s)
                wait_in(s)
                _detile_block(tins[s], r2s[s], r3s[s], BLK)

                nxt = j + 2 + s
                @pl.when(jnp.logical_and(nxt <= ABPW, blk(nxt) < NBL))
                def _refill():
                    fire_in(nxt, s)
                fire_out(j + s, s)
        return _

    lax.fori_loop(0, (ABPW + 1) // 2, lambda i, c: step(2 * i, c), 0,
                  unroll=False)
    for s in range(2):
        @pl.when(blk(s) < NBL)
        def _drain():
            wait_out(s)

    @pl.when(wid == NW - 1)
    def _tail():
        pltpu.sync_copy(tbl_hbm.at[:, pl.ds(TAIL0, TAILN)], ttin)
        _detile_block(ttin, tr2, tr3, TAILN)
        pltpu.sync_copy(tr3, out_hbm.at[pl.ds(4 * NBL, 2)])


@functools.lru_cache(maxsize=1)
def _build_a():
    mesh = plsc.VectorSubcoreMesh(core_axis_name="c", subcore_axis_name="s")
    return pl.kernel(
        _body_a,
        out_type=jax.ShapeDtypeStruct((NUM_EMB * DIM // 1024, 8, 128),
                                      jnp.float32),
        mesh=mesh,
        scratch_types=(
            [pltpu.VMEM((DIM, BLK), jnp.float32) for _ in range(2)]
            + [pltpu.VMEM((BLK, DIM + 1), jnp.float32) for _ in range(2)]
            + [pltpu.VMEM((4, 8, 128), jnp.float32) for _ in range(2)]
            + [pltpu.VMEM((DIM, TAILN), jnp.float32),
               pltpu.VMEM((TAILN, DIM + 1), jnp.float32),
               pltpu.VMEM((2, 8, 128), jnp.float32)]
            + [pltpu.SemaphoreType.DMA for _ in range(4)]
        ),
        compiler_params=pltpu.CompilerParams(
            use_tc_tiling_on_sc=True, needs_layout_passes=False
        ),
    )


def _transpose_block(rbuf, tbuf):
    lanes = lax.iota(jnp.int32, L)
    td_lo = lanes // 8
    td_hi = td_lo + 2
    r_pat = lanes % 8

    def col(c, carry):
        v0 = rbuf[c, pl.ds(0, L)]
        v1 = rbuf[c, pl.ds(L, L)]
        cs = jnp.full((L,), c, jnp.int32)
        plsc.store_scatter(tbuf, [td_lo, r_pat, cs], v0)
        plsc.store_scatter(tbuf, [td_hi, r_pat, cs], v1)
        return carry

    lax.fori_loop(0, 128, col, 0, unroll=8)


def _body_b(tok_hbm, table_hbm, out_hbm, idx_v, r0, r1, t0, t1, gs0, gs1,
            os0, os1):
    wid = lax.axis_index("s") * NC + lax.axis_index("c")
    rbufs, tbufs = (r0, r1), (t0, t1)
    gsems, osems = (gs0, gs1), (os0, os1)
    pltpu.sync_copy(tok_hbm.at[:, pl.ds(512 * wid, 512)], idx_v)

    def idx_of(g):
        return idx_v.at[g // TBW, pl.ds(128 * (g % TBW), 128)]

    def fire_gather(g, s):
        pltpu.async_copy(table_hbm.at[idx_of(g)], rbufs[s], gsems[s])

    def wait_gather(s):
        pltpu.make_async_copy(
            table_hbm.at[pl.ds(0, 128)], rbufs[s], gsems[s]
        ).wait()

    def fire_out(g, s):
        h, tb = g // TBW, 4 * wid + (g % TBW)
        for td in range(4):
            pltpu.async_copy(
                tbufs[s].at[td, :, pl.ds(0, 128)], out_hbm.at[h, td, tb],
                osems[s],
            )

    def wait_out(s):
        for td in range(4):
            pltpu.make_async_copy(
                tbufs[s].at[td, :, pl.ds(0, 128)], out_hbm.at[0, td, 0],
                osems[s],
            ).wait()

    fire_gather(0, 0)
    fire_gather(1, 1)

    def step(g, _):
        for s in range(2):
            @pl.when(g >= 2)
            def _free():
                wait_out(s)
            wait_gather(s)
            _transpose_block(rbufs[s], tbufs[s])

            @pl.when(g + 2 + s < NBLK)
            def _refill():
                fire_gather(g + 2 + s, s)
            fire_out(g + s, s)
        return _

    lax.fori_loop(0, NBLK // 2, lambda i, c: step(2 * i, c), 0, unroll=False)
    wait_out(0)
    wait_out(1)


@functools.lru_cache(maxsize=1)
def _build_b():
    mesh = plsc.VectorSubcoreMesh(core_axis_name="c", subcore_axis_name="s")
    return pl.kernel(
        _body_b,
        out_type=jax.ShapeDtypeStruct((HIST, 4, 128, 8, 128), jnp.float32),
        mesh=mesh,
        scratch_types=(
            [pltpu.VMEM((HIST, 512), jnp.int32)]
            + [pltpu.VMEM((128, DIM), jnp.float32) for _ in range(2)]
            + [pltpu.VMEM((4, 8, 129), jnp.float32) for _ in range(2)]
            + [pltpu.SemaphoreType.DMA for _ in range(4)]
        ),
        compiler_params=pltpu.CompilerParams(
            use_tc_tiling_on_sc=False, needs_layout_passes=False
        ),
    )


def kernel(token_ids, embeddings):
    tok_t = jnp.transpose(token_ids)
    tbl_t = jnp.transpose(embeddings)
    table_lin = _build_a()(tbl_t).reshape(NUM_EMB, DIM)
    x = _build_b()(tok_t, table_lin)
    return x.transpose(2, 4, 0, 1, 3).reshape(BATCH, HIST, DIM)


# speedup vs baseline: 1.2959x; 1.2959x over previous
import functools

import jax
import jax.numpy as jnp
from jax import lax
from jax.experimental import pallas as pl
from jax.experimental.pallas import tpu as pltpu
from jax.experimental.pallas import tpu_sc as plsc

NUM_EMB = 1000000
DIM = 32
BATCH = 16384
HIST = 50
NC = 2
NS = 16
NW = NC * NS
TBW = BATCH // 128 // NW
NBLK = HIST * TBW
L = 16


def _transpose_block(rbuf, tbuf):
    lanes = lax.iota(jnp.int32, L)
    td_lo = lanes // 8
    td_hi = td_lo + 2
    r_pat = lanes % 8

    def col(c, carry):
        v0 = rbuf[c, pl.ds(0, L)]
        v1 = rbuf[c, pl.ds(L, L)]
        cs = jnp.full((L,), c, jnp.int32)
        plsc.store_scatter(tbuf, [td_lo, r_pat, cs], v0)
        plsc.store_scatter(tbuf, [td_hi, r_pat, cs], v1)
        return carry

    lax.fori_loop(0, 128, col, 0, unroll=8)


def _body(tok_hbm, table_hbm, out_hbm, idx_v, r0, r1, t0, t1, gs0, gs1,
          os0, os1):
    wid = lax.axis_index("s") * NC + lax.axis_index("c")
    rbufs, tbufs = (r0, r1), (t0, t1)
    gsems, osems = (gs0, gs1), (os0, os1)
    pltpu.sync_copy(tok_hbm.at[:, pl.ds(512 * wid, 512)], idx_v)

    def idx_of(g):
        return idx_v.at[g // TBW, pl.ds(128 * (g % TBW), 128)]

    def fire_gather(g, s):
        pltpu.async_copy(table_hbm.at[idx_of(g)], rbufs[s], gsems[s])

    def wait_gather(s):
        pltpu.make_async_copy(
            table_hbm.at[pl.ds(0, 128)], rbufs[s], gsems[s]
        ).wait()

    def fire_out(g, s):
        h, tb = g // TBW, 4 * wid + (g % TBW)
        for td in range(4):
            pltpu.async_copy(
                tbufs[s].at[td, :, pl.ds(0, 128)], out_hbm.at[h, td, tb],
                osems[s],
            )

    def wait_out(s):
        for td in range(4):
            pltpu.make_async_copy(
                tbufs[s].at[td, :, pl.ds(0, 128)], out_hbm.at[0, td, 0],
                osems[s],
            ).wait()

    fire_gather(0, 0)
    fire_gather(1, 1)

    def step(g, _):
        for s in range(2):
            @pl.when(g >= 2)
            def _free():
                wait_out(s)
            wait_gather(s)
            _transpose_block(rbufs[s], tbufs[s])

            @pl.when(g + 2 + s < NBLK)
            def _refill():
                fire_gather(g + 2 + s, s)
            fire_out(g + s, s)
        return _

    lax.fori_loop(0, NBLK // 2, lambda i, c: step(2 * i, c), 0, unroll=False)
    wait_out(0)
    wait_out(1)


@functools.lru_cache(maxsize=1)
def _build():
    mesh = plsc.VectorSubcoreMesh(core_axis_name="c", subcore_axis_name="s")
    return pl.kernel(
        _body,
        out_type=jax.ShapeDtypeStruct((HIST, 4, 128, 8, 128), jnp.float32),
        mesh=mesh,
        scratch_types=(
            [pltpu.VMEM((HIST, 512), jnp.int32)]
            + [pltpu.VMEM((128, DIM), jnp.float32) for _ in range(2)]
            + [pltpu.VMEM((4, 8, 129), jnp.float32) for _ in range(2)]
            + [pltpu.SemaphoreType.DMA for _ in range(4)]
        ),
        compiler_params=pltpu.CompilerParams(
            use_tc_tiling_on_sc=False, needs_layout_passes=False
        ),
    )


def kernel(token_ids, embeddings):
    tok_t = jnp.transpose(token_ids)
    x = _build()(tok_t, embeddings)
    return x.transpose(2, 4, 0, 1, 3).reshape(BATCH, HIST, DIM)
